# Initial kernel scaffold; baseline (speedup 1.0000x reference)
import jax
import jax.numpy as jnp
from jax.experimental import pallas as pl


def kernel(z_e, W):
    raise NotImplementedError("write your pallas kernel here")


# speedup vs baseline: 3.6800x; 3.6800x over previous
import functools

import jax
import jax.numpy as jnp
from jax import lax
from jax.experimental import pallas as pl
from jax.experimental.pallas import tpu as pltpu
from jax.experimental.pallas import tpu_sc as plsc

NUM_EMB = 8192
DIM = 64
BETA = 0.25

BM = 256

NUM_SC = 2
NUM_SUBCORES = 16
NW = NUM_SC * NUM_SUBCORES


def _dist_argmin_body(z_ref, wt_ref, zs_ref, ws_ref, idx_ref, acc_ref):
    mm = lax.dot_general(
        z_ref[...], wt_ref[...], (((1,), (0,)), ((), ())),
        preferred_element_type=jnp.float32)
    d = (zs_ref[...] + ws_ref[...][None, :]) - 2.0 * mm
    m = jnp.min(d, axis=1)
    iota = lax.broadcasted_iota(jnp.int32, d.shape, 1)
    sel = jnp.where(d == m[:, None], iota, jnp.int32(NUM_EMB))
    idx_ref[...] = jnp.min(sel, axis=1)

    @pl.when(pl.program_id(0) == 0)
    def _():
        acc_ref[...] = jnp.zeros((1, 1), jnp.float32)

    acc_ref[...] += jnp.sum(m).reshape(1, 1)


def _dist_argmin(z_flat, w_t, zsum, wsum):
    n = z_flat.shape[0]
    grid = n // BM
    return pl.pallas_call(
        _dist_argmin_body,
        grid=(grid,),
        in_specs=[
            pl.BlockSpec((BM, DIM), lambda i: (i, 0)),
            pl.BlockSpec((DIM, NUM_EMB), lambda i: (0, 0)),
            pl.BlockSpec((BM, 1), lambda i: (i, 0)),
            pl.BlockSpec((NUM_EMB,), lambda i: (0,)),
        ],
        out_specs=[
            pl.BlockSpec((BM,), lambda i: (i,)),
            pl.BlockSpec((1, 1), lambda i: (0, 0)),
        ],
        out_shape=[
            jax.ShapeDtypeStruct((n,), jnp.int32),
            jax.ShapeDtypeStruct((1, 1), jnp.float32),
        ],
    )(z_flat, w_t, zsum, wsum)


def _make_sc_gather(n):
    b_per_w = n // NW
    mesh = plsc.VectorSubcoreMesh(core_axis_name="c", subcore_axis_name="s")

    @functools.partial(
        pl.kernel,
        mesh=mesh,
        compiler_params=pltpu.CompilerParams(use_tc_tiling_on_sc=False),
        out_type=jax.ShapeDtypeStruct((n, DIM), jnp.float32),
        scratch_types=[
            pltpu.VMEM((b_per_w,), jnp.int32),
            pltpu.VMEM((b_per_w, DIM), jnp.float32),
            pltpu.SemaphoreType.DMA,
        ],
    )
    def gather_rows(table_hbm, idx_hbm, out_hbm, idx_v, rows_v, sem):
        wid = lax.axis_index("s") * NUM_SC + lax.axis_index("c")
        base = wid * b_per_w
        pltpu.sync_copy(idx_hbm.at[pl.ds(base, b_per_w)], idx_v)
        pltpu.async_copy(table_hbm.at[idx_v], rows_v, sem).wait()
        pltpu.sync_copy(rows_v, out_hbm.at[pl.ds(base, b_per_w)])

    return gather_rows


def kernel(z_e, W):
    B, D, H, Wd = z_e.shape
    z_flat = jnp.transpose(z_e, (0, 2, 3, 1)).reshape(-1, D)
    zsum = jnp.sum(z_flat ** 2, axis=1, keepdims=True)
    wsum = jnp.sum(W ** 2, axis=1)
    idx, dist_total = _dist_argmin(z_flat, W.T, zsum, wsum)

    z_q_flat = _make_sc_gather(z_flat.shape[0])(W, idx)

    z_q = jnp.transpose(z_q_flat.reshape(B, H, Wd, D), (0, 3, 1, 2))
    m = dist_total[0, 0] / jnp.float32(z_e.size)
    loss = BETA * m + m
    z_q_out = z_e + (z_q - z_e)
    return (z_q_out, loss)


# speedup vs baseline: 4.2351x; 1.1508x over previous
import functools

import jax
import jax.numpy as jnp
from jax import lax
from jax.experimental import pallas as pl
from jax.experimental.pallas import tpu as pltpu
from jax.experimental.pallas import tpu_sc as plsc

NUM_EMB = 8192
DIM = 64
BETA = 0.25

BM = 512

NUM_SC = 2
NUM_SUBCORES = 16
NW = NUM_SC * NUM_SUBCORES


def _dist_argmin_body(z_ref, wt_ref, zs_ref, ws_ref, idx_ref, acc_ref):
    mm = lax.dot_general(
        z_ref[...], wt_ref[...], (((1,), (0,)), ((), ())),
        preferred_element_type=jnp.float32)
    d = (zs_ref[...] + ws_ref[...][None, :]) - 2.0 * mm
    m = jnp.min(d, axis=1)
    iota = lax.broadcasted_iota(jnp.int32, d.shape, 1).astype(jnp.float32)
    sel = jnp.where(d == m[:, None], iota, jnp.float32(NUM_EMB))
    idx_ref[...] = jnp.min(sel, axis=1).astype(jnp.int32)

    @pl.when(pl.program_id(0) == 0)
    def _():
        acc_ref[...] = jnp.zeros((1, 1), jnp.float32)

    acc_ref[...] += jnp.sum(m).reshape(1, 1)


def _dist_argmin(z_flat, w_t, zsum, wsum):
    n = z_flat.shape[0]
    grid = n // BM
    return pl.pallas_call(
        _dist_argmin_body,
        grid=(grid,),
        in_specs=[
            pl.BlockSpec((BM, DIM), lambda i: (i, 0)),
            pl.BlockSpec((DIM, NUM_EMB), lambda i: (0, 0)),
            pl.BlockSpec((BM, 1), lambda i: (i, 0)),
            pl.BlockSpec((NUM_EMB,), lambda i: (0,)),
        ],
        out_specs=[
            pl.BlockSpec((BM,), lambda i: (i,)),
            pl.BlockSpec((1, 1), lambda i: (0, 0)),
        ],
        out_shape=[
            jax.ShapeDtypeStruct((n,), jnp.int32),
            jax.ShapeDtypeStruct((1, 1), jnp.float32),
        ],
    )(z_flat, w_t, zsum, wsum)


def _make_sc_gather(n):
    b_per_w = n // NW
    mesh = plsc.VectorSubcoreMesh(core_axis_name="c", subcore_axis_name="s")

    @functools.partial(
        pl.kernel,
        mesh=mesh,
        compiler_params=pltpu.CompilerParams(use_tc_tiling_on_sc=False),
        out_type=jax.ShapeDtypeStruct((n, DIM), jnp.float32),
        scratch_types=[
            pltpu.VMEM((b_per_w,), jnp.int32),
            pltpu.VMEM((b_per_w, DIM), jnp.float32),
            pltpu.SemaphoreType.DMA,
        ],
    )
    def gather_rows(table_hbm, idx_hbm, out_hbm, idx_v, rows_v, sem):
        wid = lax.axis_index("s") * NUM_SC + lax.axis_index("c")
        base = wid * b_per_w
        pltpu.sync_copy(idx_hbm.at[pl.ds(base, b_per_w)], idx_v)
        pltpu.async_copy(table_hbm.at[idx_v], rows_v, sem).wait()
        pltpu.sync_copy(rows_v, out_hbm.at[pl.ds(base, b_per_w)])

    return gather_rows


def kernel(z_e, W):
    B, D, H, Wd = z_e.shape
    z_flat = jnp.transpose(z_e, (0, 2, 3, 1)).reshape(-1, D)
    zsum = jnp.sum(z_flat ** 2, axis=1, keepdims=True)
    wsum = jnp.sum(W ** 2, axis=1)
    idx, dist_total = _dist_argmin(z_flat, W.T, zsum, wsum)

    z_q_flat = _make_sc_gather(z_flat.shape[0])(W, idx)

    z_q = jnp.transpose(z_q_flat.reshape(B, H, Wd, D), (0, 3, 1, 2))
    m = dist_total[0, 0] / jnp.float32(z_e.size)
    loss = BETA * m + m
    z_q_out = z_e + (z_q - z_e)
    return (z_q_out, loss)
